# Initial kernel scaffold; baseline (speedup 1.0000x reference)
import jax
import jax.numpy as jnp
from jax.experimental import pallas as pl


def kernel(x, adjs, W0, b0, W1, b1):
    raise NotImplementedError("write your pallas kernel here")


# speedup vs baseline: 5.3278x; 5.3278x over previous
import functools

import jax
import jax.numpy as jnp
from jax import lax
from jax.experimental import pallas as pl
from jax.experimental.pallas import tpu as pltpu
from jax.experimental.pallas import tpu_sc as plsc

N_NODES = 10000
N_EDGES = 320000
NFEAT = 128
NHID = 128
NCLASS = 64

ROW_BLK = 1000
N_GRID = N_NODES // ROW_BLK

C = 128
CHUNKS_PER_TILE = 160
N_CHUNKS = 16 * CHUNKS_PER_TILE
E_PAD = N_CHUNKS * C
ROWS_PER_TILE = 632
N_PAD = 16 * ROWS_PER_TILE


def _mm0_body(x_ref, w_ref, o_ref):
    o_ref[0] = jnp.dot(x_ref[...], w_ref[0],
                       preferred_element_type=jnp.float32)


def _matmul0(x, W0s):
    return pl.pallas_call(
        _mm0_body,
        grid=(N_GRID, 2),
        in_specs=[
            pl.BlockSpec((ROW_BLK, NFEAT), lambda i, j: (i, 0)),
            pl.BlockSpec((1, NFEAT, NHID // 2), lambda i, j: (j, 0, 0)),
        ],
        out_specs=pl.BlockSpec((1, ROW_BLK, NHID // 2), lambda i, j: (j, i, 0)),
        out_shape=jax.ShapeDtypeStruct((2, N_NODES, NHID // 2), jnp.float32),
    )(x, W0s)


def _fuse1_body(p_ref, b_ref, w_ref, o_ref):
    z = jnp.concatenate([p_ref[0], p_ref[1]], axis=1) + b_ref[...]
    h = jnp.maximum(z, 0.0)
    o_ref[0] = jnp.dot(h, w_ref[0], preferred_element_type=jnp.float32)


def _fuse1(p0, b0, W1s):
    return pl.pallas_call(
        _fuse1_body,
        grid=(N_GRID, 2),
        in_specs=[
            pl.BlockSpec((2, ROW_BLK, NHID // 2), lambda i, j: (0, i, 0)),
            pl.BlockSpec((1, NHID), lambda i, j: (0, 0)),
            pl.BlockSpec((1, NHID, NCLASS // 2), lambda i, j: (j, 0, 0)),
        ],
        out_specs=pl.BlockSpec((1, ROW_BLK, NCLASS // 2),
                               lambda i, j: (j, i, 0)),
        out_shape=jax.ShapeDtypeStruct((2, N_NODES, NCLASS // 2), jnp.float32),
    )(p0, b0, W1s)


def _fuse2_body(p_ref, b_ref, o_ref):
    z = jnp.concatenate([p_ref[0], p_ref[1]], axis=1) + b_ref[...]
    m = jnp.max(z, axis=1, keepdims=True)
    e = jnp.exp(z - m)
    s = jnp.sum(e, axis=1, keepdims=True)
    o_ref[...] = z - m - jnp.log(s)


def _fuse2(p1, b1):
    return pl.pallas_call(
        _fuse2_body,
        grid=(N_GRID,),
        in_specs=[
            pl.BlockSpec((2, ROW_BLK, NCLASS // 2), lambda i: (0, i, 0)),
            pl.BlockSpec((1, NCLASS), lambda i: (0, 0)),
        ],
        out_specs=pl.BlockSpec((ROW_BLK, NCLASS), lambda i: (i, 0)),
        out_shape=jax.ShapeDtypeStruct((N_NODES, NCLASS), jnp.float32),
    )(p1, b1)


def _make_agg(D):
    mesh = plsc.VectorSubcoreMesh(core_axis_name="c", subcore_axis_name="s")

    @functools.partial(
        pl.kernel,
        out_type=jax.ShapeDtypeStruct((2, N_PAD, D), jnp.float32),
        mesh=mesh,
        scratch_types=[
            pltpu.VMEM((CHUNKS_PER_TILE, C), jnp.int32),
            pltpu.VMEM((CHUNKS_PER_TILE, C), jnp.int32),
            pltpu.VMEM((C, D), jnp.float32),
            pltpu.VMEM((C, D), jnp.float32),
            pltpu.VMEM_SHARED((N_PAD, D), jnp.float32),
            pltpu.SemaphoreType.DMA,
            pltpu.SemaphoreType.DMA,
            pltpu.SemaphoreType.DMA,
        ],
        compiler_params=pltpu.CompilerParams(use_tc_tiling_on_sc=False),
    )
    def agg(src_hbm, dst_hbm, table_hbm, zeros_hbm, out_hbm,
            src_v, dst_v, buf_a, buf_b, acc, sem_a, sem_b, sem_i):
        cid = lax.axis_index("c")
        sid = lax.axis_index("s")
        chunk0 = sid * CHUNKS_PER_TILE

        cp_s = pltpu.async_copy(
            src_hbm.at[cid, pl.ds(chunk0, CHUNKS_PER_TILE)], src_v, sem_i)
        cp_d = pltpu.async_copy(
            dst_hbm.at[pl.ds(chunk0, CHUNKS_PER_TILE)], dst_v, sem_i)
        pltpu.sync_copy(zeros_hbm,
                        acc.at[pl.ds(sid * ROWS_PER_TILE, ROWS_PER_TILE)])
        cp_s.wait()
        cp_d.wait()
        plsc.subcore_barrier()

        def gather_start(j, buf, sem):
            pltpu.async_copy(table_hbm.at[src_v.at[j]], buf, sem)

        def gather_wait(j, buf, sem):
            pltpu.make_async_copy(table_hbm.at[src_v.at[j]], buf, sem).wait()

        def scat_add(j, buf):
            pltpu.sync_copy(buf, acc.at[dst_v.at[j]], add=True)

        gather_start(0, buf_a, sem_a)

        def body(k, carry):
            j0 = 2 * k
            j1 = j0 + 1
            gather_start(j1, buf_b, sem_b)
            gather_wait(j0, buf_a, sem_a)
            scat_add(j0, buf_a)
            gather_start(j0 + 2, buf_a, sem_a)
            gather_wait(j1, buf_b, sem_b)
            scat_add(j1, buf_b)
            return carry

        lax.fori_loop(0, CHUNKS_PER_TILE // 2 - 1, body, 0)
        jl = CHUNKS_PER_TILE - 2
        gather_start(jl + 1, buf_b, sem_b)
        gather_wait(jl, buf_a, sem_a)
        scat_add(jl, buf_a)
        gather_wait(jl + 1, buf_b, sem_b)
        scat_add(jl + 1, buf_b)

        plsc.subcore_barrier()
        pltpu.sync_copy(
            acc.at[pl.ds(sid * ROWS_PER_TILE, ROWS_PER_TILE)],
            out_hbm.at[cid, pl.ds(sid * ROWS_PER_TILE, ROWS_PER_TILE)])

    return agg


_agg_h = _make_agg(NHID // 2)
_agg_c = _make_agg(NCLASS // 2)


@jax.jit
def kernel(x, adjs, W0, b0, W1, b1):
    adjs = adjs.astype(jnp.int32)
    pad = E_PAD - N_EDGES
    src = jnp.concatenate([adjs[0], jnp.zeros((pad,), jnp.int32)])
    src = jnp.stack([src, src + N_NODES]).reshape(2, N_CHUNKS, C)
    dst = jnp.concatenate(
        [adjs[1], jnp.full((pad,), N_NODES, jnp.int32)]).reshape(N_CHUNKS, C)

    zeros_h = jnp.zeros((ROWS_PER_TILE, NHID // 2), jnp.float32)
    zeros_c = jnp.zeros((ROWS_PER_TILE, NCLASS // 2), jnp.float32)

    W0s = jnp.stack([W0[:, :NHID // 2], W0[:, NHID // 2:]])
    W1s = jnp.stack([W1[:, :NCLASS // 2], W1[:, NCLASS // 2:]])

    support0 = _matmul0(x, W0s).reshape(2 * N_NODES, NHID // 2)
    p0 = _agg_h(src, dst, support0, zeros_h)
    support1 = _fuse1(p0, b0.reshape(1, NHID), W1s)
    support1 = support1.reshape(2 * N_NODES, NCLASS // 2)
    p1 = _agg_c(src, dst, support1, zeros_c)
    return _fuse2(p1, b1.reshape(1, NCLASS))


# speedup vs baseline: 5.3888x; 1.0115x over previous
import functools

import jax
import jax.numpy as jnp
from jax import lax
from jax.experimental import pallas as pl
from jax.experimental.pallas import tpu as pltpu
from jax.experimental.pallas import tpu_sc as plsc

N_NODES = 10000
N_EDGES = 320000
NFEAT = 128
NHID = 128
NCLASS = 64

ROW_BLK = 1000
N_GRID = N_NODES // ROW_BLK

C = 128
CHUNKS_PER_TILE = 160
N_CHUNKS = 16 * CHUNKS_PER_TILE
E_PAD = N_CHUNKS * C
ROWS_PER_TILE = 632
N_PAD = 16 * ROWS_PER_TILE


def _mm0_body(x_ref, w_ref, o_ref):
    o_ref[0] = jnp.dot(x_ref[...], w_ref[0],
                       preferred_element_type=jnp.float32)


def _matmul0(x, W0s):
    return pl.pallas_call(
        _mm0_body,
        grid=(N_GRID, 2),
        in_specs=[
            pl.BlockSpec((ROW_BLK, NFEAT), lambda i, j: (i, 0)),
            pl.BlockSpec((1, NFEAT, NHID // 2), lambda i, j: (j, 0, 0)),
        ],
        out_specs=pl.BlockSpec((1, ROW_BLK, NHID // 2), lambda i, j: (j, i, 0)),
        out_shape=jax.ShapeDtypeStruct((2, N_NODES, NHID // 2), jnp.float32),
    )(x, W0s)


def _fuse1_body(p_ref, b_ref, w_ref, o_ref):
    z = jnp.concatenate([p_ref[0], p_ref[1]], axis=1) + b_ref[...]
    h = jnp.maximum(z, 0.0)
    o_ref[0] = jnp.dot(h, w_ref[0], preferred_element_type=jnp.float32)


def _fuse1(p0, b0, W1s):
    return pl.pallas_call(
        _fuse1_body,
        grid=(N_GRID, 2),
        in_specs=[
            pl.BlockSpec((2, ROW_BLK, NHID // 2), lambda i, j: (0, i, 0)),
            pl.BlockSpec((1, NHID), lambda i, j: (0, 0)),
            pl.BlockSpec((1, NHID, NCLASS // 2), lambda i, j: (j, 0, 0)),
        ],
        out_specs=pl.BlockSpec((1, ROW_BLK, NCLASS // 2),
                               lambda i, j: (j, i, 0)),
        out_shape=jax.ShapeDtypeStruct((2, N_NODES, NCLASS // 2), jnp.float32),
    )(p0, b0, W1s)


def _fuse2_body(p_ref, b_ref, o_ref):
    z = jnp.concatenate([p_ref[0], p_ref[1]], axis=1) + b_ref[...]
    m = jnp.max(z, axis=1, keepdims=True)
    e = jnp.exp(z - m)
    s = jnp.sum(e, axis=1, keepdims=True)
    o_ref[...] = z - m - jnp.log(s)


def _fuse2(p1, b1):
    return pl.pallas_call(
        _fuse2_body,
        grid=(N_GRID,),
        in_specs=[
            pl.BlockSpec((2, ROW_BLK, NCLASS // 2), lambda i: (0, i, 0)),
            pl.BlockSpec((1, NCLASS), lambda i: (0, 0)),
        ],
        out_specs=pl.BlockSpec((ROW_BLK, NCLASS), lambda i: (i, 0)),
        out_shape=jax.ShapeDtypeStruct((N_NODES, NCLASS), jnp.float32),
    )(p1, b1)


def _make_agg(D):
    mesh = plsc.VectorSubcoreMesh(core_axis_name="c", subcore_axis_name="s")

    @functools.partial(
        pl.kernel,
        out_type=jax.ShapeDtypeStruct((2, N_PAD, D), jnp.float32),
        mesh=mesh,
        scratch_types=[
            pltpu.VMEM((CHUNKS_PER_TILE, C), jnp.int32),
            pltpu.VMEM((CHUNKS_PER_TILE, C), jnp.int32),
            [pltpu.VMEM((C, D), jnp.float32)] * 4,
            pltpu.VMEM_SHARED((N_PAD, D), jnp.float32),
            [pltpu.SemaphoreType.DMA] * 4,
            [pltpu.SemaphoreType.DMA] * 4,
            pltpu.SemaphoreType.DMA,
        ],
        compiler_params=pltpu.CompilerParams(use_tc_tiling_on_sc=False),
    )
    def agg(src_hbm, dst_hbm, table_hbm, zeros_hbm, out_hbm,
            src_v, dst_v, bufs, acc, gsem, ssem, sem_i):
        cid = lax.axis_index("c")
        sid = lax.axis_index("s")
        chunk0 = sid * CHUNKS_PER_TILE

        cp_s = pltpu.async_copy(
            src_hbm.at[cid, pl.ds(chunk0, CHUNKS_PER_TILE)], src_v, sem_i)
        cp_d = pltpu.async_copy(
            dst_hbm.at[pl.ds(chunk0, CHUNKS_PER_TILE)], dst_v, sem_i)
        pltpu.sync_copy(zeros_hbm,
                        acc.at[pl.ds(sid * ROWS_PER_TILE, ROWS_PER_TILE)])
        cp_s.wait()
        cp_d.wait()
        plsc.subcore_barrier()

        def gather_start(j, b):
            pltpu.async_copy(table_hbm.at[src_v.at[j]], bufs[b], gsem[b])

        def gather_wait(j, b):
            pltpu.make_async_copy(
                table_hbm.at[src_v.at[j]], bufs[b], gsem[b]).wait()

        def scat_start(j, b):
            pltpu.async_copy(bufs[b], acc.at[dst_v.at[j]], ssem[b], add=True)

        def scat_wait(j, b):
            pltpu.make_async_copy(
                bufs[b], acc.at[dst_v.at[j]], ssem[b]).wait()

        n = CHUNKS_PER_TILE
        gather_start(0, 0)
        gather_start(1, 1)
        gather_wait(0, 0)
        scat_start(0, 0)
        gather_start(2, 2)
        gather_wait(1, 1)
        scat_start(1, 1)
        gather_start(3, 3)

        def body(m, carry):
            for t in range(4):
                j = 4 * m + 2 + t
                b = (2 + t) % 4
                gather_wait(j, b)
                scat_start(j, b)
                scat_wait(j - 2, t % 4)
                gather_start(j + 2, t % 4)
            return carry

        lax.fori_loop(0, (n - 4) // 4, body, 0)
        gather_wait(n - 2, (n - 2) % 4)
        scat_start(n - 2, (n - 2) % 4)
        gather_wait(n - 1, (n - 1) % 4)
        scat_start(n - 1, (n - 1) % 4)
        for j in range(n - 4, n):
            scat_wait(j, j % 4)

        plsc.subcore_barrier()
        pltpu.sync_copy(
            acc.at[pl.ds(sid * ROWS_PER_TILE, ROWS_PER_TILE)],
            out_hbm.at[cid, pl.ds(sid * ROWS_PER_TILE, ROWS_PER_TILE)])

    return agg


_agg_h = _make_agg(NHID // 2)
_agg_c = _make_agg(NCLASS // 2)


@jax.jit
def kernel(x, adjs, W0, b0, W1, b1):
    adjs = adjs.astype(jnp.int32)
    pad = E_PAD - N_EDGES
    src = jnp.concatenate([adjs[0], jnp.zeros((pad,), jnp.int32)])
    src = jnp.stack([src, src + N_NODES]).reshape(2, N_CHUNKS, C)
    dst = jnp.concatenate(
        [adjs[1], jnp.full((pad,), N_NODES, jnp.int32)]).reshape(N_CHUNKS, C)

    zeros_h = jnp.zeros((ROWS_PER_TILE, NHID // 2), jnp.float32)
    zeros_c = jnp.zeros((ROWS_PER_TILE, NCLASS // 2), jnp.float32)

    W0s = jnp.stack([W0[:, :NHID // 2], W0[:, NHID // 2:]])
    W1s = jnp.stack([W1[:, :NCLASS // 2], W1[:, NCLASS // 2:]])

    support0 = _matmul0(x, W0s).reshape(2 * N_NODES, NHID // 2)
    p0 = _agg_h(src, dst, support0, zeros_h)
    support1 = _fuse1(p0, b0.reshape(1, NHID), W1s)
    support1 = support1.reshape(2 * N_NODES, NCLASS // 2)
    p1 = _agg_c(src, dst, support1, zeros_c)
    return _fuse2(p1, b1.reshape(1, NCLASS))


# speedup vs baseline: 5.3951x; 1.0012x over previous
import functools

import jax
import jax.numpy as jnp
from jax import lax
from jax.experimental import pallas as pl
from jax.experimental.pallas import tpu as pltpu
from jax.experimental.pallas import tpu_sc as plsc

N_NODES = 10000
N_EDGES = 320000
NFEAT = 128
NHID = 128
NCLASS = 64

ROW_BLK = 1000
N_GRID = N_NODES // ROW_BLK

C = 128
E_PAD = 327680
N_CHUNKS = E_PAD // C
CHUNKS_PER_TILE = N_CHUNKS // 16
ROWS_PER_TILE = 632
N_PAD = 16 * ROWS_PER_TILE


def _mm0_body(x_ref, w_ref, o_ref):
    o_ref[0] = jnp.dot(x_ref[...], w_ref[0],
                       preferred_element_type=jnp.float32)


def _matmul0(x, W0s):
    return pl.pallas_call(
        _mm0_body,
        grid=(N_GRID, 2),
        in_specs=[
            pl.BlockSpec((ROW_BLK, NFEAT), lambda i, j: (i, 0)),
            pl.BlockSpec((1, NFEAT, NHID // 2), lambda i, j: (j, 0, 0)),
        ],
        out_specs=pl.BlockSpec((1, ROW_BLK, NHID // 2), lambda i, j: (j, i, 0)),
        out_shape=jax.ShapeDtypeStruct((2, N_NODES, NHID // 2), jnp.float32),
    )(x, W0s)


def _fuse1_body(p_ref, b_ref, w_ref, o_ref):
    z = jnp.concatenate([p_ref[0], p_ref[1]], axis=1) + b_ref[...]
    h = jnp.maximum(z, 0.0)
    o_ref[0] = jnp.dot(h, w_ref[0], preferred_element_type=jnp.float32)


def _fuse1(p0, b0, W1s):
    return pl.pallas_call(
        _fuse1_body,
        grid=(N_GRID, 2),
        in_specs=[
            pl.BlockSpec((2, ROW_BLK, NHID // 2), lambda i, j: (0, i, 0)),
            pl.BlockSpec((1, NHID), lambda i, j: (0, 0)),
            pl.BlockSpec((1, NHID, NCLASS // 2), lambda i, j: (j, 0, 0)),
        ],
        out_specs=pl.BlockSpec((1, ROW_BLK, NCLASS // 2),
                               lambda i, j: (j, i, 0)),
        out_shape=jax.ShapeDtypeStruct((2, N_NODES, NCLASS // 2), jnp.float32),
    )(p0, b0, W1s)


def _fuse2_body(p_ref, b_ref, o_ref):
    z = jnp.concatenate([p_ref[0], p_ref[1]], axis=1) + b_ref[...]
    m = jnp.max(z, axis=1, keepdims=True)
    e = jnp.exp(z - m)
    s = jnp.sum(e, axis=1, keepdims=True)
    o_ref[...] = z - m - jnp.log(s)


def _fuse2(p1, b1):
    return pl.pallas_call(
        _fuse2_body,
        grid=(N_GRID,),
        in_specs=[
            pl.BlockSpec((2, ROW_BLK, NCLASS // 2), lambda i: (0, i, 0)),
            pl.BlockSpec((1, NCLASS), lambda i: (0, 0)),
        ],
        out_specs=pl.BlockSpec((ROW_BLK, NCLASS), lambda i: (i, 0)),
        out_shape=jax.ShapeDtypeStruct((N_NODES, NCLASS), jnp.float32),
    )(p1, b1)


def _make_agg(D):
    mesh = plsc.VectorSubcoreMesh(core_axis_name="c", subcore_axis_name="s")

    @functools.partial(
        pl.kernel,
        out_type=jax.ShapeDtypeStruct((2, N_PAD, D), jnp.float32),
        mesh=mesh,
        scratch_types=[
            pltpu.VMEM((CHUNKS_PER_TILE, C), jnp.int32),
            pltpu.VMEM((CHUNKS_PER_TILE, C), jnp.int32),
            [pltpu.VMEM((C, D), jnp.float32)] * 4,
            pltpu.VMEM_SHARED((N_PAD, D), jnp.float32),
            [pltpu.SemaphoreType.DMA] * 4,
            [pltpu.SemaphoreType.DMA] * 4,
            pltpu.SemaphoreType.DMA,
        ],
        compiler_params=pltpu.CompilerParams(use_tc_tiling_on_sc=False),
    )
    def agg(src_hbm, dst_hbm, table_hbm, zeros_hbm, out_hbm,
            src_v, dst_v, bufs, acc, gsem, ssem, sem_i):
        cid = lax.axis_index("c")
        sid = lax.axis_index("s")
        chunk0 = sid * CHUNKS_PER_TILE

        cp_s = pltpu.async_copy(
            src_hbm.at[cid, pl.ds(chunk0, CHUNKS_PER_TILE)], src_v, sem_i)
        cp_d = pltpu.async_copy(
            dst_hbm.at[pl.ds(chunk0, CHUNKS_PER_TILE)], dst_v, sem_i)
        pltpu.sync_copy(zeros_hbm,
                        acc.at[pl.ds(sid * ROWS_PER_TILE, ROWS_PER_TILE)])
        cp_s.wait()
        cp_d.wait()
        plsc.subcore_barrier()

        def gather_start(j, b):
            pltpu.async_copy(table_hbm.at[src_v.at[j]], bufs[b], gsem[b])

        def gather_wait(j, b):
            pltpu.make_async_copy(
                table_hbm.at[src_v.at[j]], bufs[b], gsem[b]).wait()

        def scat_start(j, b):
            pltpu.async_copy(bufs[b], acc.at[dst_v.at[j]], ssem[b], add=True)

        def scat_wait(j, b):
            pltpu.make_async_copy(
                bufs[b], acc.at[dst_v.at[j]], ssem[b]).wait()

        n = CHUNKS_PER_TILE
        gather_start(0, 0)
        gather_start(1, 1)
        gather_wait(0, 0)
        scat_start(0, 0)
        gather_start(2, 2)
        gather_wait(1, 1)
        scat_start(1, 1)
        gather_start(3, 3)

        def body(m, carry):
            for t in range(4):
                j = 4 * m + 2 + t
                b = (2 + t) % 4
                gather_wait(j, b)
                scat_start(j, b)
                scat_wait(j - 2, t % 4)
                gather_start(j + 2, t % 4)
            return carry

        lax.fori_loop(0, (n - 4) // 4, body, 0)
        gather_wait(n - 2, (n - 2) % 4)
        scat_start(n - 2, (n - 2) % 4)
        gather_wait(n - 1, (n - 1) % 4)
        scat_start(n - 1, (n - 1) % 4)
        for j in range(n - 4, n):
            scat_wait(j, j % 4)

        plsc.subcore_barrier()
        pltpu.sync_copy(
            acc.at[pl.ds(sid * ROWS_PER_TILE, ROWS_PER_TILE)],
            out_hbm.at[cid, pl.ds(sid * ROWS_PER_TILE, ROWS_PER_TILE)])

    return agg


_agg_h = _make_agg(NHID // 2)
_agg_c = _make_agg(NCLASS // 2)


@jax.jit
def kernel(x, adjs, W0, b0, W1, b1):
    adjs = adjs.astype(jnp.int32)
    pad = E_PAD - N_EDGES
    src = jnp.pad(adjs[0], (0, pad))
    src = jnp.stack([src, src + N_NODES]).reshape(2, N_CHUNKS, C)
    dst = jnp.pad(adjs[1], (0, pad),
                  constant_values=N_NODES).reshape(N_CHUNKS, C)

    zeros_h = jnp.zeros((ROWS_PER_TILE, NHID // 2), jnp.float32)
    zeros_c = jnp.zeros((ROWS_PER_TILE, NCLASS // 2), jnp.float32)

    W0s = jnp.stack([W0[:, :NHID // 2], W0[:, NHID // 2:]])
    W1s = jnp.stack([W1[:, :NCLASS // 2], W1[:, NCLASS // 2:]])

    support0 = _matmul0(x, W0s).reshape(2 * N_NODES, NHID // 2)
    p0 = _agg_h(src, dst, support0, zeros_h)
    support1 = _fuse1(p0, b0.reshape(1, NHID), W1s)
    support1 = support1.reshape(2 * N_NODES, NCLASS // 2)
    p1 = _agg_c(src, dst, support1, zeros_c)
    return _fuse2(p1, b1.reshape(1, NCLASS))


# speedup vs baseline: 5.5815x; 1.0345x over previous
import functools

import jax
import jax.numpy as jnp
from jax import lax
from jax.experimental import pallas as pl
from jax.experimental.pallas import tpu as pltpu
from jax.experimental.pallas import tpu_sc as plsc

N_NODES = 10000
N_EDGES = 320000
NFEAT = 128
NHID = 128
NCLASS = 64

ROW_BLK = 1000
N_GRID = N_NODES // ROW_BLK

C = 128
E_PAD = 327680
N_CHUNKS = E_PAD // C
CHUNKS_PER_TILE = N_CHUNKS // 16
ROWS_PER_TILE = 632
N_PAD = 16 * ROWS_PER_TILE


def _mm0_body(x_ref, w_ref, o_ref):
    o_ref[0] = jnp.dot(x_ref[...], w_ref[0],
                       preferred_element_type=jnp.float32)


def _matmul0(x, W0s):
    return pl.pallas_call(
        _mm0_body,
        grid=(N_GRID, 2),
        in_specs=[
            pl.BlockSpec((ROW_BLK, NFEAT), lambda i, j: (i, 0)),
            pl.BlockSpec((1, NFEAT, NHID // 2), lambda i, j: (j, 0, 0)),
        ],
        out_specs=pl.BlockSpec((1, ROW_BLK, NHID // 2), lambda i, j: (j, i, 0)),
        out_shape=jax.ShapeDtypeStruct((2, N_NODES, NHID // 2), jnp.float32),
    )(x, W0s)


def _fuse1_body(p_ref, b_ref, w_ref, o_ref):
    z = jnp.concatenate([p_ref[0], p_ref[1]], axis=1) + b_ref[...]
    h = jnp.maximum(z, 0.0)
    o_ref[0] = jnp.dot(h, w_ref[0], preferred_element_type=jnp.float32)


def _fuse1(p0, b0, W1s):
    return pl.pallas_call(
        _fuse1_body,
        grid=(N_GRID, 2),
        in_specs=[
            pl.BlockSpec((2, ROW_BLK, NHID // 2), lambda i, j: (0, i, 0)),
            pl.BlockSpec((1, NHID), lambda i, j: (0, 0)),
            pl.BlockSpec((1, NHID, NCLASS // 2), lambda i, j: (j, 0, 0)),
        ],
        out_specs=pl.BlockSpec((1, ROW_BLK, NCLASS // 2),
                               lambda i, j: (j, i, 0)),
        out_shape=jax.ShapeDtypeStruct((2, N_NODES, NCLASS // 2), jnp.float32),
    )(p0, b0, W1s)


def _fuse2_body(p_ref, b_ref, o_ref):
    z = jnp.concatenate([p_ref[0], p_ref[1]], axis=1) + b_ref[...]
    m = jnp.max(z, axis=1, keepdims=True)
    e = jnp.exp(z - m)
    s = jnp.sum(e, axis=1, keepdims=True)
    o_ref[...] = z - m - jnp.log(s)


def _fuse2(p1, b1):
    return pl.pallas_call(
        _fuse2_body,
        grid=(N_GRID,),
        in_specs=[
            pl.BlockSpec((2, ROW_BLK, NCLASS // 2), lambda i: (0, i, 0)),
            pl.BlockSpec((1, NCLASS), lambda i: (0, 0)),
        ],
        out_specs=pl.BlockSpec((ROW_BLK, NCLASS), lambda i: (i, 0)),
        out_shape=jax.ShapeDtypeStruct((N_NODES, NCLASS), jnp.float32),
    )(p1, b1)


def _make_agg(D):
    mesh = plsc.VectorSubcoreMesh(core_axis_name="c", subcore_axis_name="s")

    @functools.partial(
        pl.kernel,
        out_type=jax.ShapeDtypeStruct((2, N_PAD, D), jnp.float32),
        mesh=mesh,
        scratch_types=[
            pltpu.VMEM((CHUNKS_PER_TILE, C), jnp.int32),
            pltpu.VMEM((CHUNKS_PER_TILE, C), jnp.int32),
            [pltpu.VMEM((C, D), jnp.float32)] * 6,
            pltpu.VMEM_SHARED((N_PAD, D), jnp.float32),
            [pltpu.SemaphoreType.DMA] * 6,
            [pltpu.SemaphoreType.DMA] * 6,
            pltpu.SemaphoreType.DMA,
        ],
        compiler_params=pltpu.CompilerParams(use_tc_tiling_on_sc=False),
    )
    def agg(src_hbm, dst_hbm, table_hbm, zeros_hbm, out_hbm,
            src_v, dst_v, bufs, acc, gsem, ssem, sem_i):
        cid = lax.axis_index("c")
        sid = lax.axis_index("s")
        chunk0 = sid * CHUNKS_PER_TILE

        cp_s = pltpu.async_copy(
            src_hbm.at[cid, pl.ds(chunk0, CHUNKS_PER_TILE)], src_v, sem_i)
        cp_d = pltpu.async_copy(
            dst_hbm.at[pl.ds(chunk0, CHUNKS_PER_TILE)], dst_v, sem_i)
        pltpu.sync_copy(zeros_hbm,
                        acc.at[pl.ds(sid * ROWS_PER_TILE, ROWS_PER_TILE)])
        cp_s.wait()
        cp_d.wait()
        plsc.subcore_barrier()

        def gather_start(j, b):
            pltpu.async_copy(table_hbm.at[src_v.at[j]], bufs[b], gsem[b])

        def gather_wait(j, b):
            pltpu.make_async_copy(
                table_hbm.at[src_v.at[j]], bufs[b], gsem[b]).wait()

        def scat_start(j, b):
            pltpu.async_copy(bufs[b], acc.at[dst_v.at[j]], ssem[b], add=True)

        def scat_wait(j, b):
            pltpu.make_async_copy(
                bufs[b], acc.at[dst_v.at[j]], ssem[b]).wait()

        n = CHUNKS_PER_TILE
        for j in range(4):
            gather_start(j, j)
        for j in range(2):
            gather_wait(j, j)
            scat_start(j, j)
            gather_start(j + 4, j + 4)

        def body(m, carry):
            for t in range(6):
                j = 6 * m + 2 + t
                b = (2 + t) % 6
                gather_wait(j, b)
                scat_start(j, b)
                scat_wait(j - 2, t % 6)
                gather_start(j + 4, t % 6)
            return carry

        n_steady = (n - 6) // 6
        lax.fori_loop(0, n_steady, body, 0)
        for j in range(6 * n_steady + 2, n):
            b = j % 6
            gather_wait(j, b)
            scat_start(j, b)
            scat_wait(j - 2, (j - 2) % 6)
            if j + 4 < n:
                gather_start(j + 4, (j - 2) % 6)
        scat_wait(n - 2, (n - 2) % 6)
        scat_wait(n - 1, (n - 1) % 6)

        plsc.subcore_barrier()
        pltpu.sync_copy(
            acc.at[pl.ds(sid * ROWS_PER_TILE, ROWS_PER_TILE)],
            out_hbm.at[cid, pl.ds(sid * ROWS_PER_TILE, ROWS_PER_TILE)])

    return agg


_agg_h = _make_agg(NHID // 2)
_agg_c = _make_agg(NCLASS // 2)


@jax.jit
def kernel(x, adjs, W0, b0, W1, b1):
    adjs = adjs.astype(jnp.int32)
    pad = E_PAD - N_EDGES
    src = jnp.pad(adjs[0], (0, pad))
    src = jnp.stack([src, src + N_NODES]).reshape(2, N_CHUNKS, C)
    dst = jnp.pad(adjs[1], (0, pad),
                  constant_values=N_NODES).reshape(N_CHUNKS, C)

    zeros_h = jnp.zeros((ROWS_PER_TILE, NHID // 2), jnp.float32)
    zeros_c = jnp.zeros((ROWS_PER_TILE, NCLASS // 2), jnp.float32)

    W0s = jnp.stack([W0[:, :NHID // 2], W0[:, NHID // 2:]])
    W1s = jnp.stack([W1[:, :NCLASS // 2], W1[:, NCLASS // 2:]])

    support0 = _matmul0(x, W0s).reshape(2 * N_NODES, NHID // 2)
    p0 = _agg_h(src, dst, support0, zeros_h)
    support1 = _fuse1(p0, b0.reshape(1, NHID), W1s)
    support1 = support1.reshape(2 * N_NODES, NCLASS // 2)
    p1 = _agg_c(src, dst, support1, zeros_c)
    return _fuse2(p1, b1.reshape(1, NCLASS))


# speedup vs baseline: 7.0247x; 1.2586x over previous
import functools

import jax
import jax.numpy as jnp
import numpy as np
from jax import lax
from jax.experimental import pallas as pl
from jax.experimental.pallas import tpu as pltpu
from jax.experimental.pallas import tpu_sc as plsc

N_NODES = 10000
N_EDGES = 320000
NFEAT = 128
NHID = 128
NCLASS = 64

MM_BLK = 2000
MM_GRID = N_NODES // MM_BLK
ROW_BLK = 1000
N_GRID = N_NODES // ROW_BLK

C = 128
E_PAD = 327680
N_CHUNKS = E_PAD // C
CHUNKS_PER_TILE = N_CHUNKS // 16
ROWS_PER_TILE = 632
N_PAD = 16 * ROWS_PER_TILE

_I16 = np.arange(16)
_PERM32 = np.stack([_I16, _I16 + 16], axis=1).ravel()
_PERM64 = np.concatenate([_PERM32, _PERM32 + 32])


def _mm0_body(x_ref, w_ref, o_ref):
    o_ref[0] = jnp.dot(x_ref[...], w_ref[0],
                       preferred_element_type=jnp.float32
                       ).astype(jnp.bfloat16)


def _matmul0(x, W0s):
    return pl.pallas_call(
        _mm0_body,
        grid=(MM_GRID, 2),
        in_specs=[
            pl.BlockSpec((MM_BLK, NFEAT), lambda i, j: (i, 0)),
            pl.BlockSpec((1, NFEAT, NHID // 2), lambda i, j: (j, 0, 0)),
        ],
        out_specs=pl.BlockSpec((1, MM_BLK, NHID // 2), lambda i, j: (j, i, 0)),
        out_shape=jax.ShapeDtypeStruct((2, N_NODES, NHID // 2), jnp.bfloat16),
    )(x, W0s)


def _fuse1_body(p_ref, b_ref, w_ref, o_ref):
    z = jnp.concatenate([p_ref[0], p_ref[1]], axis=1) + b_ref[...]
    h = jnp.maximum(z, 0.0)
    o_ref[0] = jnp.dot(h, w_ref[0], preferred_element_type=jnp.float32
                       ).astype(jnp.bfloat16)


def _fuse1(p0, b0, W1s):
    return pl.pallas_call(
        _fuse1_body,
        grid=(MM_GRID, 2),
        in_specs=[
            pl.BlockSpec((2, MM_BLK, NHID // 2), lambda i, j: (0, i, 0)),
            pl.BlockSpec((1, NHID), lambda i, j: (0, 0)),
            pl.BlockSpec((1, NHID, NCLASS // 2), lambda i, j: (j, 0, 0)),
        ],
        out_specs=pl.BlockSpec((1, MM_BLK, NCLASS // 2),
                               lambda i, j: (j, i, 0)),
        out_shape=jax.ShapeDtypeStruct((2, N_NODES, NCLASS // 2),
                                       jnp.bfloat16),
    )(p0, b0, W1s)


def _fuse2_body(p_ref, b_ref, o_ref):
    z = jnp.concatenate([p_ref[0], p_ref[1]], axis=1) + b_ref[...]
    m = jnp.max(z, axis=1, keepdims=True)
    e = jnp.exp(z - m)
    s = jnp.sum(e, axis=1, keepdims=True)
    o_ref[...] = z - m - jnp.log(s)


def _fuse2(p1, b1):
    return pl.pallas_call(
        _fuse2_body,
        grid=(N_GRID,),
        in_specs=[
            pl.BlockSpec((2, ROW_BLK, NCLASS // 2), lambda i: (0, i, 0)),
            pl.BlockSpec((1, NCLASS), lambda i: (0, 0)),
        ],
        out_specs=pl.BlockSpec((ROW_BLK, NCLASS), lambda i: (i, 0)),
        out_shape=jax.ShapeDtypeStruct((N_NODES, NCLASS), jnp.float32),
    )(p1, b1)


def _make_agg(D):
    mesh = plsc.VectorSubcoreMesh(core_axis_name="c", subcore_axis_name="s")

    @functools.partial(
        pl.kernel,
        out_type=jax.ShapeDtypeStruct((2, N_PAD, D), jnp.float32),
        mesh=mesh,
        scratch_types=[
            pltpu.VMEM((CHUNKS_PER_TILE, C), jnp.int32),
            pltpu.VMEM((CHUNKS_PER_TILE, C), jnp.int32),
            [pltpu.VMEM((C, D), jnp.bfloat16)] * 6,
            [pltpu.VMEM((C, D), jnp.float32)] * 3,
            pltpu.VMEM_SHARED((N_PAD, D), jnp.float32),
            [pltpu.SemaphoreType.DMA] * 6,
            [pltpu.SemaphoreType.DMA] * 3,
            pltpu.SemaphoreType.DMA,
        ],
        compiler_params=pltpu.CompilerParams(use_tc_tiling_on_sc=False,
                                             needs_layout_passes=False),
    )
    def agg(src_hbm, dst_hbm, table_hbm, zeros_hbm, out_hbm,
            src_v, dst_v, bbufs, fbufs, acc, gsem, ssem, sem_i):
        cid = lax.axis_index("c")
        sid = lax.axis_index("s")
        chunk0 = sid * CHUNKS_PER_TILE

        cp_s = pltpu.async_copy(
            src_hbm.at[cid, pl.ds(chunk0, CHUNKS_PER_TILE)], src_v, sem_i)
        cp_d = pltpu.async_copy(
            dst_hbm.at[pl.ds(chunk0, CHUNKS_PER_TILE)], dst_v, sem_i)
        pltpu.sync_copy(zeros_hbm,
                        acc.at[pl.ds(sid * ROWS_PER_TILE, ROWS_PER_TILE)])
        cp_s.wait()
        cp_d.wait()
        plsc.subcore_barrier()

        def gather_start(j, b):
            pltpu.async_copy(table_hbm.at[src_v.at[j]], bbufs[b], gsem[b])

        def gather_wait(j, b):
            pltpu.make_async_copy(
                table_hbm.at[src_v.at[j]], bbufs[b], gsem[b]).wait()

        def scat_start(j, s):
            pltpu.async_copy(fbufs[s], acc.at[dst_v.at[j]], ssem[s], add=True)

        def scat_wait(j, s):
            pltpu.make_async_copy(
                fbufs[s], acc.at[dst_v.at[j]], ssem[s]).wait()

        def conv(b, s):
            bf = bbufs[b]
            f32 = fbufs[s]

            def crow(r, carry):
                row_bf = bf.at[r]
                row_f = f32.at[r]
                for g in range(D // 32):
                    lo, hi = plsc.unpack(
                        row_bf[pl.ds(32 * g, 32)],
                        format=plsc.PackFormat.INTERLEAVED)
                    row_f[pl.ds(32 * g, 16)] = lo
                    row_f[pl.ds(32 * g + 16, 16)] = hi
                return carry

            lax.fori_loop(0, C, crow, 0)

        n = CHUNKS_PER_TILE
        for j in range(4):
            gather_start(j, j)
        for j in range(3):
            gather_wait(j, j)
            conv(j, j)
            scat_start(j, j)
            gather_start(j + 4, (j + 4) % 6)

        def body(m, carry):
            for t in range(6):
                j = 6 * m + 3 + t
                bb = (3 + t) % 6
                fs = (3 + t) % 3
                gather_wait(j, bb)
                scat_wait(j - 3, fs)
                conv(bb, fs)
                scat_start(j, fs)
                gather_start(j + 4, (3 + t + 4) % 6)
            return carry

        n_steady = (n - 6) // 6
        lax.fori_loop(0, n_steady, body, 0)
        for j in range(6 * n_steady + 3, n):
            bb = j % 6
            fs = j % 3
            gather_wait(j, bb)
            scat_wait(j - 3, fs)
            conv(bb, fs)
            scat_start(j, fs)
            if j + 4 < n:
                gather_start(j + 4, (j + 4) % 6)
        for j in range(n - 3, n):
            scat_wait(j, j % 3)

        plsc.subcore_barrier()
        pltpu.sync_copy(
            acc.at[pl.ds(sid * ROWS_PER_TILE, ROWS_PER_TILE)],
            out_hbm.at[cid, pl.ds(sid * ROWS_PER_TILE, ROWS_PER_TILE)])

    return agg


_agg_h = _make_agg(NHID // 2)
_agg_c = _make_agg(NCLASS // 2)


@jax.jit
def kernel(x, adjs, W0, b0, W1, b1):
    adjs = adjs.astype(jnp.int32)
    pad = E_PAD - N_EDGES
    src = jnp.pad(adjs[0], (0, pad))
    src = jnp.stack([src, src + N_NODES]).reshape(2, N_CHUNKS, C)
    dst = jnp.pad(adjs[1], (0, pad),
                  constant_values=N_NODES).reshape(N_CHUNKS, C)

    zeros_h = jnp.zeros((ROWS_PER_TILE, NHID // 2), jnp.float32)
    zeros_c = jnp.zeros((ROWS_PER_TILE, NCLASS // 2), jnp.float32)

    W0s = jnp.stack([W0[:, :NHID // 2][:, _PERM64],
                     W0[:, NHID // 2:][:, _PERM64]])
    W1s = jnp.stack([W1[:, :NCLASS // 2][:, _PERM32],
                     W1[:, NCLASS // 2:][:, _PERM32]])

    support0 = _matmul0(x, W0s).reshape(2 * N_NODES, NHID // 2)
    p0 = _agg_h(src, dst, support0, zeros_h)
    support1 = _fuse1(p0, b0.reshape(1, NHID), W1s)
    support1 = support1.reshape(2 * N_NODES, NCLASS // 2)
    p1 = _agg_c(src, dst, support1, zeros_c)
    return _fuse2(p1, b1.reshape(1, NCLASS))


# speedup vs baseline: 7.6447x; 1.0883x over previous
import functools

import jax
import jax.numpy as jnp
import numpy as np
from jax import lax
from jax.experimental import pallas as pl
from jax.experimental.pallas import tpu as pltpu
from jax.experimental.pallas import tpu_sc as plsc

N_NODES = 10000
N_EDGES = 320000
NFEAT = 128
NHID = 128
NCLASS = 64

MM_BLK = 2000
MM_GRID = N_NODES // MM_BLK
ROW_BLK = 1000
N_GRID = N_NODES // ROW_BLK

C = 128
E_PAD = 327680
N_CHUNKS = E_PAD // C
CHUNKS_PER_TILE = N_CHUNKS // 16
ROWS_PER_TILE = 632
N_PAD = 16 * ROWS_PER_TILE

_I16 = np.arange(16)
_PERM32 = np.stack([_I16, _I16 + 16], axis=1).ravel()
_PERM64 = np.concatenate([_PERM32, _PERM32 + 32])


def _mm0_body(x_ref, w_ref, o_ref):
    o_ref[0] = jnp.dot(x_ref[...], w_ref[0],
                       preferred_element_type=jnp.float32
                       ).astype(jnp.bfloat16)


def _matmul0(x, W0s):
    return pl.pallas_call(
        _mm0_body,
        grid=(MM_GRID, 2),
        in_specs=[
            pl.BlockSpec((MM_BLK, NFEAT), lambda i, j: (i, 0)),
            pl.BlockSpec((1, NFEAT, NHID // 2), lambda i, j: (j, 0, 0)),
        ],
        out_specs=pl.BlockSpec((1, MM_BLK, NHID // 2), lambda i, j: (j, i, 0)),
        out_shape=jax.ShapeDtypeStruct((2, N_NODES, NHID // 2), jnp.bfloat16),
    )(x, W0s)


def _fuse1_body(p_ref, b_ref, w_ref, o_ref):
    z = jnp.concatenate([p_ref[0], p_ref[1]], axis=1) + b_ref[...]
    h = jnp.maximum(z, 0.0)
    o_ref[0] = jnp.dot(h, w_ref[0], preferred_element_type=jnp.float32
                       ).astype(jnp.bfloat16)


def _fuse1(p0, b0, W1s):
    return pl.pallas_call(
        _fuse1_body,
        grid=(MM_GRID, 2),
        in_specs=[
            pl.BlockSpec((2, MM_BLK, NHID // 2), lambda i, j: (0, i, 0)),
            pl.BlockSpec((1, NHID), lambda i, j: (0, 0)),
            pl.BlockSpec((1, NHID, NCLASS // 2), lambda i, j: (j, 0, 0)),
        ],
        out_specs=pl.BlockSpec((1, MM_BLK, NCLASS // 2),
                               lambda i, j: (j, i, 0)),
        out_shape=jax.ShapeDtypeStruct((2, N_NODES, NCLASS // 2),
                                       jnp.bfloat16),
    )(p0, b0, W1s)


def _fuse2_body(p_ref, b_ref, o_ref):
    z = jnp.concatenate([p_ref[0], p_ref[1]],
                        axis=1).astype(jnp.float32) + b_ref[...]
    m = jnp.max(z, axis=1, keepdims=True)
    e = jnp.exp(z - m)
    s = jnp.sum(e, axis=1, keepdims=True)
    o_ref[...] = z - m - jnp.log(s)


def _fuse2(p1, b1):
    return pl.pallas_call(
        _fuse2_body,
        grid=(MM_GRID,),
        in_specs=[
            pl.BlockSpec((2, MM_BLK, NCLASS // 2), lambda i: (0, i, 0)),
            pl.BlockSpec((1, NCLASS), lambda i: (0, 0)),
        ],
        out_specs=pl.BlockSpec((MM_BLK, NCLASS), lambda i: (i, 0)),
        out_shape=jax.ShapeDtypeStruct((N_NODES, NCLASS), jnp.float32),
    )(p1, b1)


def _make_agg(D):
    mesh = plsc.VectorSubcoreMesh(core_axis_name="c", subcore_axis_name="s")

    @functools.partial(
        pl.kernel,
        out_type=jax.ShapeDtypeStruct((2, N_PAD, D), jnp.float32),
        mesh=mesh,
        scratch_types=[
            pltpu.VMEM((CHUNKS_PER_TILE, C), jnp.int32),
            pltpu.VMEM((CHUNKS_PER_TILE, C), jnp.int32),
            [pltpu.VMEM((C, D), jnp.bfloat16)] * 6,
            [pltpu.VMEM((C, D), jnp.float32)] * 3,
            pltpu.VMEM_SHARED((N_PAD, D), jnp.float32),
            [pltpu.SemaphoreType.DMA] * 6,
            [pltpu.SemaphoreType.DMA] * 3,
            pltpu.SemaphoreType.DMA,
        ],
        compiler_params=pltpu.CompilerParams(use_tc_tiling_on_sc=False,
                                             needs_layout_passes=False),
    )
    def agg(src_hbm, dst_hbm, table_hbm, zeros_hbm, out_hbm,
            src_v, dst_v, bbufs, fbufs, acc, gsem, ssem, sem_i):
        cid = lax.axis_index("c")
        sid = lax.axis_index("s")
        chunk0 = sid * CHUNKS_PER_TILE

        cp_s = pltpu.async_copy(
            src_hbm.at[cid, pl.ds(chunk0, CHUNKS_PER_TILE)], src_v, sem_i)
        cp_d = pltpu.async_copy(
            dst_hbm.at[pl.ds(chunk0, CHUNKS_PER_TILE)], dst_v, sem_i)
        pltpu.sync_copy(zeros_hbm,
                        acc.at[pl.ds(sid * ROWS_PER_TILE, ROWS_PER_TILE)])
        cp_s.wait()
        cp_d.wait()
        plsc.subcore_barrier()

        def gather_start(j, b):
            pltpu.async_copy(table_hbm.at[src_v.at[j]], bbufs[b], gsem[b])

        def gather_wait(j, b):
            pltpu.make_async_copy(
                table_hbm.at[src_v.at[j]], bbufs[b], gsem[b]).wait()

        def scat_start(j, s):
            pltpu.async_copy(fbufs[s], acc.at[dst_v.at[j]], ssem[s], add=True)

        def scat_wait(j, s):
            pltpu.make_async_copy(
                fbufs[s], acc.at[dst_v.at[j]], ssem[s]).wait()

        def conv(b, s):
            bf = bbufs[b]
            f32 = fbufs[s]

            def crow(r, carry):
                row_bf = bf.at[r]
                row_f = f32.at[r]
                for g in range(D // 32):
                    lo, hi = plsc.unpack(
                        row_bf[pl.ds(32 * g, 32)],
                        format=plsc.PackFormat.INTERLEAVED)
                    row_f[pl.ds(32 * g, 16)] = lo
                    row_f[pl.ds(32 * g + 16, 16)] = hi
                return carry

            lax.fori_loop(0, C, crow, 0)

        n = CHUNKS_PER_TILE
        for j in range(4):
            gather_start(j, j)
        for j in range(3):
            gather_wait(j, j)
            conv(j, j)
            scat_start(j, j)
            gather_start(j + 4, (j + 4) % 6)

        def body(m, carry):
            for t in range(6):
                j = 6 * m + 3 + t
                bb = (3 + t) % 6
                fs = (3 + t) % 3
                gather_wait(j, bb)
                scat_wait(j - 3, fs)
                conv(bb, fs)
                scat_start(j, fs)
                gather_start(j + 4, (3 + t + 4) % 6)
            return carry

        n_steady = (n - 6) // 6
        lax.fori_loop(0, n_steady, body, 0)
        for j in range(6 * n_steady + 3, n):
            bb = j % 6
            fs = j % 3
            gather_wait(j, bb)
            scat_wait(j - 3, fs)
            conv(bb, fs)
            scat_start(j, fs)
            if j + 4 < n:
                gather_start(j + 4, (j + 4) % 6)
        for j in range(n - 3, n):
            scat_wait(j, j % 3)

        plsc.subcore_barrier()
        pltpu.sync_copy(
            acc.at[pl.ds(sid * ROWS_PER_TILE, ROWS_PER_TILE)],
            out_hbm.at[cid, pl.ds(sid * ROWS_PER_TILE, ROWS_PER_TILE)])

    return agg


def _make_agg_bf(D):
    mesh = plsc.VectorSubcoreMesh(core_axis_name="c", subcore_axis_name="s")

    @functools.partial(
        pl.kernel,
        out_type=jax.ShapeDtypeStruct((2, N_PAD, D), jnp.bfloat16),
        mesh=mesh,
        scratch_types=[
            pltpu.VMEM((CHUNKS_PER_TILE, C), jnp.int32),
            pltpu.VMEM((CHUNKS_PER_TILE, C), jnp.int32),
            [pltpu.VMEM((C, D), jnp.bfloat16)] * 6,
            pltpu.VMEM_SHARED((N_PAD, D), jnp.bfloat16),
            [pltpu.SemaphoreType.DMA] * 6,
            [pltpu.SemaphoreType.DMA] * 6,
            pltpu.SemaphoreType.DMA,
        ],
        compiler_params=pltpu.CompilerParams(use_tc_tiling_on_sc=False,
                                             needs_layout_passes=False),
    )
    def agg(src_hbm, dst_hbm, table_hbm, zeros_hbm, out_hbm,
            src_v, dst_v, bufs, acc, gsem, ssem, sem_i):
        cid = lax.axis_index("c")
        sid = lax.axis_index("s")
        chunk0 = sid * CHUNKS_PER_TILE

        cp_s = pltpu.async_copy(
            src_hbm.at[cid, pl.ds(chunk0, CHUNKS_PER_TILE)], src_v, sem_i)
        cp_d = pltpu.async_copy(
            dst_hbm.at[pl.ds(chunk0, CHUNKS_PER_TILE)], dst_v, sem_i)
        pltpu.sync_copy(zeros_hbm,
                        acc.at[pl.ds(sid * ROWS_PER_TILE, ROWS_PER_TILE)])
        cp_s.wait()
        cp_d.wait()
        plsc.subcore_barrier()

        def gather_start(j, b):
            pltpu.async_copy(table_hbm.at[src_v.at[j]], bufs[b], gsem[b])

        def gather_wait(j, b):
            pltpu.make_async_copy(
                table_hbm.at[src_v.at[j]], bufs[b], gsem[b]).wait()

        def scat_start(j, b):
            pltpu.async_copy(bufs[b], acc.at[dst_v.at[j]], ssem[b], add=True)

        def scat_wait(j, b):
            pltpu.make_async_copy(
                bufs[b], acc.at[dst_v.at[j]], ssem[b]).wait()

        n = CHUNKS_PER_TILE
        for j in range(4):
            gather_start(j, j)
        for j in range(2):
            gather_wait(j, j)
            scat_start(j, j)
            gather_start(j + 4, j + 4)

        def body(m, carry):
            for t in range(6):
                j = 6 * m + 2 + t
                b = (2 + t) % 6
                gather_wait(j, b)
                scat_start(j, b)
                scat_wait(j - 2, t % 6)
                gather_start(j + 4, t % 6)
            return carry

        n_steady = (n - 6) // 6
        lax.fori_loop(0, n_steady, body, 0)
        for j in range(6 * n_steady + 2, n):
            b = j % 6
            gather_wait(j, b)
            scat_start(j, b)
            scat_wait(j - 2, (j - 2) % 6)
            if j + 4 < n:
                gather_start(j + 4, (j - 2) % 6)
        scat_wait(n - 2, (n - 2) % 6)
        scat_wait(n - 1, (n - 1) % 6)

        plsc.subcore_barrier()
        pltpu.sync_copy(
            acc.at[pl.ds(sid * ROWS_PER_TILE, ROWS_PER_TILE)],
            out_hbm.at[cid, pl.ds(sid * ROWS_PER_TILE, ROWS_PER_TILE)])

    return agg


_agg_h = _make_agg(NHID // 2)
_agg_c = _make_agg_bf(NCLASS // 2)


@jax.jit
def kernel(x, adjs, W0, b0, W1, b1):
    adjs = adjs.astype(jnp.int32)
    pad = E_PAD - N_EDGES
    src = jnp.pad(adjs[0], (0, pad))
    src = jnp.stack([src, src + N_NODES]).reshape(2, N_CHUNKS, C)
    dst = jnp.pad(adjs[1], (0, pad),
                  constant_values=N_NODES).reshape(N_CHUNKS, C)

    zeros_h = jnp.zeros((ROWS_PER_TILE, NHID // 2), jnp.float32)
    zeros_c = jnp.zeros((ROWS_PER_TILE, NCLASS // 2), jnp.bfloat16)

    W0s = jnp.stack([W0[:, :NHID // 2][:, _PERM64],
                     W0[:, NHID // 2:][:, _PERM64]])
    W1s = jnp.stack([W1[:, :NCLASS // 2], W1[:, NCLASS // 2:]])

    support0 = _matmul0(x, W0s).reshape(2 * N_NODES, NHID // 2)
    p0 = _agg_h(src, dst, support0, zeros_h)
    support1 = _fuse1(p0, b0.reshape(1, NHID), W1s)
    support1 = support1.reshape(2 * N_NODES, NCLASS // 2)
    p1 = _agg_c(src, dst, support1, zeros_c)
    return _fuse2(p1, b1.reshape(1, NCLASS))


# speedup vs baseline: 8.5176x; 1.1142x over previous
import functools

import jax
import jax.numpy as jnp
import numpy as np
from jax import lax
from jax.experimental import pallas as pl
from jax.experimental.pallas import tpu as pltpu
from jax.experimental.pallas import tpu_sc as plsc

N_NODES = 10000
N_EDGES = 320000
NFEAT = 128
NHID = 128
NCLASS = 64

MM_BLK = 2000
MM_GRID = N_NODES // MM_BLK
ROW_BLK = 1000
N_GRID = N_NODES // ROW_BLK

C = 128
E_PAD = 327680
N_CHUNKS = E_PAD // C
CHUNKS_PER_TILE = N_CHUNKS // 16
ROWS_PER_TILE = 632
N_PAD = 16 * ROWS_PER_TILE

_I16 = np.arange(16)
_PERM32 = np.stack([_I16, _I16 + 16], axis=1).ravel()
_PERM64 = np.concatenate([_PERM32, _PERM32 + 32])


def _mm0_body(x_ref, w_ref, o_ref):
    o_ref[0] = jnp.dot(x_ref[...], w_ref[0],
                       preferred_element_type=jnp.float32
                       ).astype(jnp.bfloat16)


def _matmul0(x, W0s):
    return pl.pallas_call(
        _mm0_body,
        grid=(MM_GRID, 2),
        in_specs=[
            pl.BlockSpec((MM_BLK, NFEAT), lambda i, j: (i, 0)),
            pl.BlockSpec((1, NFEAT, NHID // 2), lambda i, j: (j, 0, 0)),
        ],
        out_specs=pl.BlockSpec((1, MM_BLK, NHID // 2), lambda i, j: (j, i, 0)),
        out_shape=jax.ShapeDtypeStruct((2, N_NODES, NHID // 2), jnp.bfloat16),
    )(x, W0s)


def _fuse1_body(p_ref, b_ref, w_ref, o_ref):
    z = jnp.concatenate([p_ref[0], p_ref[1]], axis=1) + b_ref[...]
    h = jnp.maximum(z, 0.0)
    o_ref[0] = jnp.dot(h, w_ref[0], preferred_element_type=jnp.float32
                       ).astype(jnp.bfloat16)


def _fuse1(p0, b0, W1s):
    return pl.pallas_call(
        _fuse1_body,
        grid=(MM_GRID, 2),
        in_specs=[
            pl.BlockSpec((2, MM_BLK, NHID // 2), lambda i, j: (0, i, 0)),
            pl.BlockSpec((1, NHID), lambda i, j: (0, 0)),
            pl.BlockSpec((1, NHID, NCLASS // 2), lambda i, j: (j, 0, 0)),
        ],
        out_specs=pl.BlockSpec((1, MM_BLK, NCLASS // 2),
                               lambda i, j: (j, i, 0)),
        out_shape=jax.ShapeDtypeStruct((2, N_NODES, NCLASS // 2),
                                       jnp.bfloat16),
    )(p0, b0, W1s)


def _fuse2_body(p_ref, b_ref, o_ref):
    z = jnp.concatenate([p_ref[0], p_ref[1]],
                        axis=1).astype(jnp.float32) + b_ref[...]
    m = jnp.max(z, axis=1, keepdims=True)
    e = jnp.exp(z - m)
    s = jnp.sum(e, axis=1, keepdims=True)
    o_ref[...] = z - m - jnp.log(s)


def _fuse2(p1, b1):
    return pl.pallas_call(
        _fuse2_body,
        grid=(MM_GRID,),
        in_specs=[
            pl.BlockSpec((2, MM_BLK, NCLASS // 2), lambda i: (0, i, 0)),
            pl.BlockSpec((1, NCLASS), lambda i: (0, 0)),
        ],
        out_specs=pl.BlockSpec((MM_BLK, NCLASS), lambda i: (i, 0)),
        out_shape=jax.ShapeDtypeStruct((N_NODES, NCLASS), jnp.float32),
    )(p1, b1)


def _make_agg(D):
    mesh = plsc.VectorSubcoreMesh(core_axis_name="c", subcore_axis_name="s")

    @functools.partial(
        pl.kernel,
        out_type=jax.ShapeDtypeStruct((2, N_PAD, D), jnp.float32),
        mesh=mesh,
        scratch_types=[
            pltpu.VMEM((CHUNKS_PER_TILE, C), jnp.int32),
            pltpu.VMEM((CHUNKS_PER_TILE, C), jnp.int32),
            [pltpu.VMEM((C, D), jnp.bfloat16)] * 6,
            [pltpu.VMEM((C, D), jnp.float32)] * 3,
            pltpu.VMEM_SHARED((N_PAD, D), jnp.float32),
            [pltpu.SemaphoreType.DMA] * 6,
            [pltpu.SemaphoreType.DMA] * 3,
            pltpu.SemaphoreType.DMA,
        ],
        compiler_params=pltpu.CompilerParams(use_tc_tiling_on_sc=False,
                                             needs_layout_passes=False),
    )
    def agg(src_hbm, dst_hbm, table_hbm, zeros_hbm, out_hbm,
            src_v, dst_v, bbufs, fbufs, acc, gsem, ssem, sem_i):
        cid = lax.axis_index("c")
        sid = lax.axis_index("s")
        chunk0 = sid * CHUNKS_PER_TILE

        cp_s = pltpu.async_copy(
            src_hbm.at[cid, pl.ds(chunk0, CHUNKS_PER_TILE)], src_v, sem_i)
        cp_d = pltpu.async_copy(
            dst_hbm.at[pl.ds(chunk0, CHUNKS_PER_TILE)], dst_v, sem_i)
        pltpu.sync_copy(zeros_hbm,
                        acc.at[pl.ds(sid * ROWS_PER_TILE, ROWS_PER_TILE)])
        cp_s.wait()
        cp_d.wait()
        plsc.subcore_barrier()

        def gather_start(j, b):
            pltpu.async_copy(table_hbm.at[src_v.at[j]], bbufs[b], gsem[b])

        def gather_wait(j, b):
            pltpu.make_async_copy(
                table_hbm.at[src_v.at[j]], bbufs[b], gsem[b]).wait()

        def scat_start(j, s):
            pltpu.async_copy(fbufs[s], acc.at[dst_v.at[j]], ssem[s], add=True)

        def scat_wait(j, s):
            pltpu.make_async_copy(
                fbufs[s], acc.at[dst_v.at[j]], ssem[s]).wait()

        def conv(b, s):
            bf = bbufs[b]
            f32 = fbufs[s]

            def crow(r, carry):
                row_bf = bf.at[r]
                row_f = f32.at[r]
                for g in range(D // 32):
                    lo, hi = plsc.unpack(
                        row_bf[pl.ds(32 * g, 32)],
                        format=plsc.PackFormat.INTERLEAVED)
                    row_f[pl.ds(32 * g, 16)] = lo
                    row_f[pl.ds(32 * g + 16, 16)] = hi
                return carry

            lax.fori_loop(0, C, crow, 0)

        n = CHUNKS_PER_TILE
        for j in range(4):
            gather_start(j, j)
        for j in range(3):
            gather_wait(j, j)
            conv(j, j)
            scat_start(j, j)
            gather_start(j + 4, (j + 4) % 6)

        def body(m, carry):
            for t in range(6):
                j = 6 * m + 3 + t
                bb = (3 + t) % 6
                fs = (3 + t) % 3
                gather_wait(j, bb)
                scat_wait(j - 3, fs)
                conv(bb, fs)
                scat_start(j, fs)
                gather_start(j + 4, (3 + t + 4) % 6)
            return carry

        n_steady = (n - 6) // 6
        lax.fori_loop(0, n_steady, body, 0)
        for j in range(6 * n_steady + 3, n):
            bb = j % 6
            fs = j % 3
            gather_wait(j, bb)
            scat_wait(j - 3, fs)
            conv(bb, fs)
            scat_start(j, fs)
            if j + 4 < n:
                gather_start(j + 4, (j + 4) % 6)
        for j in range(n - 3, n):
            scat_wait(j, j % 3)

        plsc.subcore_barrier()
        pltpu.sync_copy(
            acc.at[pl.ds(sid * ROWS_PER_TILE, ROWS_PER_TILE)],
            out_hbm.at[cid, pl.ds(sid * ROWS_PER_TILE, ROWS_PER_TILE)])

    return agg


def _make_agg_bf(D):
    mesh = plsc.VectorSubcoreMesh(core_axis_name="c", subcore_axis_name="s")

    @functools.partial(
        pl.kernel,
        out_type=jax.ShapeDtypeStruct((2, N_PAD, D), jnp.bfloat16),
        mesh=mesh,
        scratch_types=[
            pltpu.VMEM((CHUNKS_PER_TILE, C), jnp.int32),
            pltpu.VMEM((CHUNKS_PER_TILE, C), jnp.int32),
            [pltpu.VMEM((C, D), jnp.bfloat16)] * 6,
            pltpu.VMEM_SHARED((N_PAD, D), jnp.bfloat16),
            pltpu.VMEM_SHARED((N_NODES, D), jnp.bfloat16),
            [pltpu.SemaphoreType.DMA] * 6,
            [pltpu.SemaphoreType.DMA] * 6,
            pltpu.SemaphoreType.DMA,
        ],
        compiler_params=pltpu.CompilerParams(use_tc_tiling_on_sc=False,
                                             needs_layout_passes=False),
    )
    def agg(src_hbm, dst_hbm, table_hbm, zeros_hbm, out_hbm,
            src_v, dst_v, bufs, acc, table_s, gsem, ssem, sem_i):
        cid = lax.axis_index("c")
        sid = lax.axis_index("s")
        chunk0 = sid * CHUNKS_PER_TILE

        cp_s = pltpu.async_copy(
            src_hbm.at[0, pl.ds(chunk0, CHUNKS_PER_TILE)], src_v, sem_i)
        cp_d = pltpu.async_copy(
            dst_hbm.at[pl.ds(chunk0, CHUNKS_PER_TILE)], dst_v, sem_i)
        trows = N_NODES // 16
        pltpu.sync_copy(
            table_hbm.at[pl.ds(cid * N_NODES + sid * trows, trows)],
            table_s.at[pl.ds(sid * trows, trows)])
        pltpu.sync_copy(zeros_hbm,
                        acc.at[pl.ds(sid * ROWS_PER_TILE, ROWS_PER_TILE)])
        cp_s.wait()
        cp_d.wait()
        plsc.subcore_barrier()

        def gather_start(j, b):
            pltpu.async_copy(table_s.at[src_v.at[j]], bufs[b], gsem[b])

        def gather_wait(j, b):
            pltpu.make_async_copy(
                table_s.at[src_v.at[j]], bufs[b], gsem[b]).wait()

        def scat_start(j, b):
            pltpu.async_copy(bufs[b], acc.at[dst_v.at[j]], ssem[b], add=True)

        def scat_wait(j, b):
            pltpu.make_async_copy(
                bufs[b], acc.at[dst_v.at[j]], ssem[b]).wait()

        n = CHUNKS_PER_TILE
        for j in range(4):
            gather_start(j, j)
        for j in range(2):
            gather_wait(j, j)
            scat_start(j, j)
            gather_start(j + 4, j + 4)

        def body(m, carry):
            for t in range(6):
                j = 6 * m + 2 + t
                b = (2 + t) % 6
                gather_wait(j, b)
                scat_start(j, b)
                scat_wait(j - 2, t % 6)
                gather_start(j + 4, t % 6)
            return carry

        n_steady = (n - 6) // 6
        lax.fori_loop(0, n_steady, body, 0)
        for j in range(6 * n_steady + 2, n):
            b = j % 6
            gather_wait(j, b)
            scat_start(j, b)
            scat_wait(j - 2, (j - 2) % 6)
            if j + 4 < n:
                gather_start(j + 4, (j - 2) % 6)
        scat_wait(n - 2, (n - 2) % 6)
        scat_wait(n - 1, (n - 1) % 6)

        plsc.subcore_barrier()
        pltpu.sync_copy(
            acc.at[pl.ds(sid * ROWS_PER_TILE, ROWS_PER_TILE)],
            out_hbm.at[cid, pl.ds(sid * ROWS_PER_TILE, ROWS_PER_TILE)])

    return agg


_agg_h = _make_agg(NHID // 2)
_agg_c = _make_agg_bf(NCLASS // 2)


@jax.jit
def kernel(x, adjs, W0, b0, W1, b1):
    adjs = adjs.astype(jnp.int32)
    pad = E_PAD - N_EDGES
    src = jnp.pad(adjs[0], (0, pad))
    src = jnp.stack([src, src + N_NODES]).reshape(2, N_CHUNKS, C)
    dst = jnp.pad(adjs[1], (0, pad),
                  constant_values=N_NODES).reshape(N_CHUNKS, C)

    zeros_h = jnp.zeros((ROWS_PER_TILE, NHID // 2), jnp.float32)
    zeros_c = jnp.zeros((ROWS_PER_TILE, NCLASS // 2), jnp.bfloat16)

    W0s = jnp.stack([W0[:, :NHID // 2][:, _PERM64],
                     W0[:, NHID // 2:][:, _PERM64]])
    W1s = jnp.stack([W1[:, :NCLASS // 2], W1[:, NCLASS // 2:]])

    support0 = _matmul0(x, W0s).reshape(2 * N_NODES, NHID // 2)
    p0 = _agg_h(src, dst, support0, zeros_h)
    support1 = _fuse1(p0, b0.reshape(1, NHID), W1s)
    support1 = support1.reshape(2 * N_NODES, NCLASS // 2)
    p1 = _agg_c(src, dst, support1, zeros_c)
    return _fuse2(p1, b1.reshape(1, NCLASS))
